# baseline (speedup 1.0000x reference)
import functools

import jax
import jax.numpy as jnp
from jax import lax
from jax.experimental import pallas as pl
from jax.experimental.pallas import tpu as pltpu
from jax.experimental.pallas import tpu_sc as plsc

_NW = 32
_CW = 128
_L = 16
_TBLK = 4096


def _fmt_rows(x_t):
    D, N = x_t.shape
    nblk = (N + _TBLK - 1) // _TBLK

    def body(in_ref, out_ref):
        eye = jnp.eye(D, dtype=jnp.float32)
        eye2 = jnp.concatenate([eye, eye], axis=1)
        out_ref[...] = jax.lax.dot_general(
            in_ref[...], eye2, (((0,), (0,)), ((), ())),
            precision=jax.lax.Precision.DEFAULT,
        )

    return pl.pallas_call(
        body,
        grid=(nblk,),
        in_specs=[pl.BlockSpec((D, _TBLK), lambda i: (0, i))],
        out_specs=pl.BlockSpec((_TBLK, 2 * D), lambda i: (i, 0)),
        out_shape=jax.ShapeDtypeStruct((N, 2 * D), jnp.float32),
    )(x_t)


def kernel(embeddings, labels, centers):
    B, D = embeddings.shape
    ch = B // (_NW * _CW)
    bw = _CW * ch
    DP = 2 * D
    nring = 2

    ctr_r = _fmt_rows(centers.T)
    idx = labels.astype(jnp.int32)

    mesh = plsc.VectorSubcoreMesh(core_axis_name="c", subcore_axis_name="s")

    @functools.partial(
        pl.kernel,
        mesh=mesh,
        compiler_params=pltpu.CompilerParams(
            use_tc_tiling_on_sc=True, needs_layout_passes=False
        ),
        out_type=jax.ShapeDtypeStruct((_NW * _L,), jnp.float32),
        scratch_types=[
            pltpu.VMEM((bw,), jnp.int32),
            pltpu.VMEM((bw, D), jnp.float32),
            pltpu.VMEM((nring * _CW, DP), jnp.float32),
            pltpu.VMEM((_L,), jnp.float32),
            pltpu.SemaphoreType.DMA,
            pltpu.SemaphoreType.DMA,
            pltpu.SemaphoreType.DMA,
        ],
    )
    def sc_kernel(emb_hbm, idx_hbm, ctr_hbm, out_hbm,
                  idx_v, emb_v, ctr_v, acc_v, sem_e, sem_g0, sem_g1):
        wid = lax.axis_index("s") * 2 + lax.axis_index("c")
        base = pl.multiple_of(wid * bw, bw)

        emb_dma = pltpu.async_copy(emb_hbm.at[pl.ds(base, bw)], emb_v, sem_e)
        pltpu.sync_copy(idx_hbm.at[pl.ds(base, bw)], idx_v)
        sems = [sem_g0, sem_g1]

        def fire(j):
            return pltpu.async_copy(
                ctr_hbm.at[idx_v.at[pl.ds(j * _CW, _CW)]],
                ctr_v.at[pl.ds((j % nring) * _CW, _CW)],
                sems[j % nring],
            )

        gathers = [fire(0), fire(1)]
        emb_dma.wait()

        ngrp = _CW // 8

        def make_grp_body(slot):
            def grp_body(g, acc):
                ebase = pl.multiple_of(g * 8, 8)
                cbase = pl.multiple_of((slot * ngrp + g % ngrp) * 8, 8)
                ev = emb_v.at[pl.ds(ebase, 8)]
                cv = ctr_v.at[pl.ds(cbase, 8)]
                for k in range(8):
                    for c in range(D // _L):
                        e = ev[k, pl.ds(c * _L, _L)]
                        t = cv[k, pl.ds(c * _L, _L)]
                        d = e - t
                        acc = acc + d * d
                return acc
            return grp_body

        acc = jnp.zeros((_L,), jnp.float32)
        for j in range(ch):
            gathers[j].wait()
            acc = lax.fori_loop(j * ngrp, (j + 1) * ngrp,
                                make_grp_body(j % nring), acc)
            if j + nring < ch:
                gathers.append(fire(j + nring))

        acc_v[...] = acc
        pltpu.sync_copy(acc_v, out_hbm.at[pl.ds(wid * _L, _L)])

    partials = sc_kernel(embeddings, idx, ctr_r)
    return partials.sum() / B
